# baseline (speedup 1.0000x reference)
import functools

import jax
import jax.numpy as jnp
from jax import lax
from jax.experimental import pallas as pl
from jax.experimental.pallas import tpu as pltpu
from jax.experimental.pallas import tpu_sc as plsc

_NC = 2
_NS = 16
_NW = _NC * _NS
_L = 16
_CHUNK = 512

_LN2 = 0.6931471805599453
_SQRT2 = 1.4142135623730951
_LOG_C = (
    -3.332947384568352e-06,
    0.9999100019104871,
    -0.49933572632078504,
    0.3376105578963719,
    -0.27109935070790736,
    0.17028616221812656,
)

_GATHER_DNUMS = lax.GatherDimensionNumbers(
    offset_dims=(), collapsed_slice_dims=(0,), start_index_map=(0,)
)


def _lane_gather(x, idx):
    return lax.gather(
        x,
        idx.reshape(_L, 1),
        _GATHER_DNUMS,
        (1,),
        mode=lax.GatherScatterMode.PROMISE_IN_BOUNDS,
    )


def _fast_log(x):
    xi = lax.bitcast_convert_type(x, jnp.int32)
    e = lax.shift_right_arithmetic(xi, 23) - 127
    m = lax.bitcast_convert_type(
        (xi & 0x007FFFFF) | 0x3F800000, jnp.float32
    )
    big = m > _SQRT2
    m = jnp.where(big, m * 0.5, m)
    ef = e.astype(jnp.float32) + jnp.where(big, 1.0, 0.0)
    z = m - 1.0
    p = jnp.float32(_LOG_C[5])
    for k in (4, 3, 2, 1, 0):
        p = p * z + jnp.float32(_LOG_C[k])
    return ef * jnp.float32(_LN2) + p


def _body(e1_hbm, e2_hbm, out_hbm, b1, b2, ob):
    wid = lax.axis_index("s") * _NC + lax.axis_index("c")
    n_elems = e1_hbm.shape[0]
    per_w = n_elems // _NW
    n_chunks = per_w // _CHUNK

    iota = lax.iota(jnp.int32, _L)
    low2 = iota & 3
    high2 = iota - low2
    idx_a = [high2 + s1 for s1 in range(4)]
    idx_b = [low2 + 4 * s1 for s1 in range(4)]

    def chunk_body(ci, _):
        el0 = wid * per_w + ci * _CHUNK
        pltpu.sync_copy(e1_hbm.at[pl.ds(el0, _CHUNK)], b1)
        pltpu.sync_copy(e2_hbm.at[pl.ds(el0, _CHUNK)], b2)

        def elem_body(i, _):
            p10 = jnp.exp(b1[i, 0])
            p11 = jnp.exp(b1[i, 1])
            p20 = jnp.exp(b2[i, 0])
            p21 = jnp.exp(b2[i, 1])
            acc0 = jnp.zeros((_L,), jnp.float32)
            acc1 = jnp.zeros((_L,), jnp.float32)
            for s1 in range(4):
                a0 = _lane_gather(p10, idx_a[s1])
                a1 = _lane_gather(p11, idx_a[s1])
                c0 = _lane_gather(p20, idx_b[s1])
                c1 = _lane_gather(p21, idx_b[s1])
                acc0 = acc0 + a0 * c0 + a1 * c1
                acc1 = acc1 + a1 * c0 + a0 * c1
            ob[i, 0] = _fast_log(acc0)
            ob[i, 1] = _fast_log(acc1)
            return ()

        lax.fori_loop(0, _CHUNK, elem_body, ())
        pltpu.sync_copy(ob, out_hbm.at[pl.ds(el0, _CHUNK)])
        return ()

    lax.fori_loop(0, n_chunks, chunk_body, ())


def kernel(e1, e2):
    b0, b1_, u, s_in, s_out = e1.shape
    n = b0 * b1_
    e1f = e1.reshape(n, 2, 16)
    e2f = e2.reshape(n, 2, 16)
    mesh = plsc.VectorSubcoreMesh(core_axis_name="c", subcore_axis_name="s")
    run = pl.kernel(
        _body,
        out_type=jax.ShapeDtypeStruct((n, 2, 16), jnp.float32),
        mesh=mesh,
        scratch_types=[
            pltpu.VMEM((_CHUNK, 2, _L), jnp.float32),
            pltpu.VMEM((_CHUNK, 2, _L), jnp.float32),
            pltpu.VMEM((_CHUNK, 2, _L), jnp.float32),
        ],
        compiler_params=pltpu.CompilerParams(use_tc_tiling_on_sc=False),
    )
    out = run(e1f, e2f)
    return out.reshape(b0, b1_, u, s_in, s_out)
